# Initial kernel scaffold; baseline (speedup 1.0000x reference)
import jax
import jax.numpy as jnp
from jax.experimental import pallas as pl


def kernel(p, x):
    raise NotImplementedError("write your pallas kernel here")


# speedup vs baseline: 2.0813x; 2.0813x over previous
import functools

import jax
import jax.numpy as jnp
import numpy as np
from jax.experimental import pallas as pl

_IN_CHANNELS = 3
_OUT_CHANNELS = 384
_FEAT_DIM = _OUT_CHANNELS // (_IN_CHANNELS * 2)
_ROWS_PER_CH = 2 * _FEAT_DIM
_ALPHA = 100.0
_WAVE = 500.0

_BLOCK_N = 2048


def _pe_kernel(scale_ref, phase_ref, x_ref, out_ref):
    scale = scale_ref[:, :]
    phase = phase_ref[:, :]
    rows = x_ref.shape[0]
    for c in range(rows):
        v = x_ref[c, :][None, :]
        out_ref[c * _ROWS_PER_CH:(c + 1) * _ROWS_PER_CH, :] = jnp.sin(
            v * scale + phase)


@jax.jit
def kernel(p, x):
    B, C, N = x.shape
    x2 = x.reshape(B * C, N)

    j = np.arange(_ROWS_PER_CH) // 2
    scale_np = _ALPHA * np.power(_WAVE, -(j.astype(np.float64) / _FEAT_DIM))
    phase_np = np.where(np.arange(_ROWS_PER_CH) % 2 == 1, np.pi / 2, 0.0)
    scale = jnp.asarray(scale_np, dtype=jnp.float32).reshape(_ROWS_PER_CH, 1)
    phase = jnp.asarray(phase_np, dtype=jnp.float32).reshape(_ROWS_PER_CH, 1)

    out_rows = B * C * _ROWS_PER_CH
    num_blocks = pl.cdiv(N, _BLOCK_N)

    out2 = pl.pallas_call(
        _pe_kernel,
        grid=(num_blocks,),
        in_specs=[
            pl.BlockSpec((_ROWS_PER_CH, 1), lambda i: (0, 0)),
            pl.BlockSpec((_ROWS_PER_CH, 1), lambda i: (0, 0)),
            pl.BlockSpec((B * C, _BLOCK_N), lambda i: (0, i)),
        ],
        out_specs=pl.BlockSpec((out_rows, _BLOCK_N), lambda i: (0, i)),
        out_shape=jax.ShapeDtypeStruct((out_rows, N), jnp.float32),
    )(scale, phase, x2)

    return (p, out2.reshape(B, C * _ROWS_PER_CH, N))


# speedup vs baseline: 4.2857x; 2.0592x over previous
import jax
import jax.numpy as jnp
import numpy as np
from jax.experimental import pallas as pl
from jax.experimental.pallas import tpu as pltpu

_IN_CHANNELS = 3
_OUT_CHANNELS = 384
_FEAT_DIM = _OUT_CHANNELS // (_IN_CHANNELS * 2)
_ROWS_PER_CH = 2 * _FEAT_DIM
_ALPHA = 100.0
_WAVE = 500.0

_BLOCK_N = 2048

_POLY = (6.283183466376198, -41.34148035624613, 81.59765787614151,
         -76.59492821657152, 41.269929567670836, -12.3724948184423)


def _pe_kernel(s2_ref, p2_ref, x_ref, out_ref):
    s2 = s2_ref[:, :]
    p2 = p2_ref[:, :]
    rows = x_ref.shape[0]
    c1, c3, c5, c7, c9, c11 = (np.float32(c) for c in _POLY)
    for c in range(rows):
        v = x_ref[c, :][None, :]
        t = v * s2 + p2
        r = t - jnp.round(t)
        u = r * r
        poly = ((((c11 * u + c9) * u + c7) * u + c5) * u + c3) * u + c1
        out_ref[c * _ROWS_PER_CH:(c + 1) * _ROWS_PER_CH, :] = poly * r


@jax.jit
def kernel(p, x):
    B, C, N = x.shape
    x2 = x.reshape(B * C, N)

    j = np.arange(_ROWS_PER_CH) // 2
    s2_np = (_ALPHA * np.power(_WAVE, -(j.astype(np.float64) / _FEAT_DIM))
             / (2.0 * np.pi))
    p2_np = np.where(np.arange(_ROWS_PER_CH) % 2 == 1, 0.25, 0.0)
    s2 = jnp.asarray(s2_np, dtype=jnp.float32).reshape(_ROWS_PER_CH, 1)
    p2 = jnp.asarray(p2_np, dtype=jnp.float32).reshape(_ROWS_PER_CH, 1)

    out_rows = B * C * _ROWS_PER_CH
    num_blocks = pl.cdiv(N, _BLOCK_N)

    out2 = pl.pallas_call(
        _pe_kernel,
        grid=(num_blocks,),
        in_specs=[
            pl.BlockSpec((_ROWS_PER_CH, 1), lambda i: (0, 0)),
            pl.BlockSpec((_ROWS_PER_CH, 1), lambda i: (0, 0)),
            pl.BlockSpec((B * C, _BLOCK_N), lambda i: (0, i)),
        ],
        out_specs=pl.BlockSpec((out_rows, _BLOCK_N), lambda i: (0, i)),
        out_shape=jax.ShapeDtypeStruct((out_rows, N), jnp.float32),
        compiler_params=pltpu.CompilerParams(
            dimension_semantics=("parallel",)),
    )(s2, p2, x2)

    return (p, out2.reshape(B, C * _ROWS_PER_CH, N))
